# baseline (speedup 1.0000x reference)
import jax
import jax.numpy as jnp
from jax import lax
from jax.experimental import pallas as pl
from jax.experimental.pallas import tpu as pltpu
from jax.experimental.pallas import tpu_sc as plsc

VOCAB = 1000000
EMB = 32
B = 4096
L = 200

NC = 2
NS = 16
NW = NC * NS
BPW = B // NW
IPW = BPW * L
S0 = 96
S1 = L - S0
W = 128
PR = VOCAB // 4
CW = 1024
NFULL = VOCAB // CW
TAILW = VOCAB - NFULL * CW
TAILBASE = NFULL * CW


def _repack_body(tt_hbm, tail_hbm, out_hbm, in_v, out_v, sem):
    c = lax.axis_index("c")
    s = lax.axis_index("s")
    wid = s * NC + c

    lane = lax.iota(jnp.int32, 16)

    def do_chunk(base, width):
        pltpu.async_copy(tt_hbm.at[:, pl.ds(base, width)],
                         in_v.at[:, pl.ds(0, width)], sem).wait()

        def col_loop(g, __):
            vl0 = g * 16

            def e_loop(e, ___):
                vals = in_v[e, pl.ds(vl0, 16)]
                plsc.store_scatter(out_v, [lane * EMB + (vl0 * EMB + e)], vals)
                return 0

            lax.fori_loop(0, EMB, e_loop, 0)
            return 0

        lax.fori_loop(0, width // 16, col_loop, 0)
        pltpu.sync_copy(out_v.at[pl.ds(0, width * EMB)],
                        out_hbm.at[pl.ds(base * EMB, width * EMB)])

    def full_chunk(i, _):
        do_chunk((wid + i * NW) * CW, CW)
        return 0

    n_i = jnp.where(wid < NFULL - (NFULL // NW) * NW, NFULL // NW + 1,
                    NFULL // NW)
    lax.fori_loop(0, n_i, full_chunk, 0)

    @pl.when(wid == 16)
    def _():
        do_chunk(TAILBASE, 512)

    @pl.when(wid == 17)
    def _():
        pltpu.sync_copy(tail_hbm,
                        out_hbm.at[pl.ds((TAILBASE + 512) * EMB, 64 * EMB)])


def _gather_body(ids_hbm, tbl_hbm, out_hbm, idx_v, buf_a, buf_b, out_v, sem_a, sem_b):
    c = lax.axis_index("c")
    s = lax.axis_index("s")
    wid = s * NC + c

    pltpu.sync_copy(ids_hbm.at[pl.ds(wid * IPW, IPW)], idx_v)

    def fire(r, buf, sem):
        pltpu.async_copy(tbl_hbm.at[idx_v.at[pl.ds(L * r, S0)]],
                         buf.at[pl.ds(0, S0)], sem)
        pltpu.async_copy(tbl_hbm.at[idx_v.at[pl.ds(L * r + S0, S1)]],
                         buf.at[pl.ds(S0, S1)], sem)

    def drain(buf, sem):
        pltpu.make_async_copy(tbl_hbm.at[pl.ds(0, L)], buf, sem).wait()

    def accum(buf, r):
        zero = jnp.zeros((16,), jnp.float32)

        def body(j, carry):
            a0, a1 = carry
            a0 = a0 + buf[j, pl.ds(0, 16)]
            a1 = a1 + buf[j, pl.ds(16, 16)]
            return a0, a1

        a0, a1 = lax.fori_loop(0, L, body, (zero, zero), unroll=8)
        out_v[pl.ds(EMB * r, 16)] = a0
        out_v[pl.ds(EMB * r + 16, 16)] = a1

    fire(0, buf_a, sem_a)
    fire(1, buf_b, sem_b)

    def step(i, _):
        g = 2 * i
        drain(buf_a, sem_a)
        accum(buf_a, g)
        fire(jnp.minimum(g + 2, BPW - 1), buf_a, sem_a)
        drain(buf_b, sem_b)
        accum(buf_b, g + 1)
        fire(jnp.minimum(g + 3, BPW - 1), buf_b, sem_b)
        return 0

    lax.fori_loop(0, BPW // 2, step, 0)
    drain(buf_a, sem_a)
    drain(buf_b, sem_b)

    pltpu.sync_copy(out_v, out_hbm.at[pl.ds(wid * BPW * EMB, BPW * EMB)])


@jax.jit
def _encode(ids1, table_t, tail_packed):
    mesh = plsc.VectorSubcoreMesh(core_axis_name="c", subcore_axis_name="s")
    repack = pl.kernel(
        _repack_body,
        out_type=jax.ShapeDtypeStruct((PR * W,), jnp.float32),
        mesh=mesh,
        scratch_types=[
            pltpu.VMEM((EMB, CW), jnp.float32),
            pltpu.VMEM((CW * EMB,), jnp.float32),
            pltpu.SemaphoreType.DMA,
        ],
        name="repack",
        compiler_params=pltpu.CompilerParams(use_tc_tiling_on_sc=True,
                                             disable_bounds_checks=True,
                                             needs_layout_passes=False),
    )
    packed = repack(table_t, tail_packed)

    gather = pl.kernel(
        _gather_body,
        out_type=jax.ShapeDtypeStruct((B * EMB,), jnp.float32),
        mesh=mesh,
        scratch_types=[
            pltpu.VMEM((IPW,), jnp.int32),
            pltpu.VMEM((L, EMB), jnp.float32),
            pltpu.VMEM((L, EMB), jnp.float32),
            pltpu.VMEM((BPW * EMB,), jnp.float32),
            pltpu.SemaphoreType.DMA,
            pltpu.SemaphoreType.DMA,
        ],
        compiler_params=pltpu.CompilerParams(use_tc_tiling_on_sc=False),
    )
    return gather(ids1, packed.reshape(VOCAB, EMB))


def kernel(input_ids, embedding_table):
    ids1 = input_ids.astype(jnp.int32).reshape(B * L)
    tail_packed = embedding_table[VOCAB - 64:, :].reshape(64 * EMB)
    out = _encode(ids1, embedding_table.T, tail_packed)
    return (out.reshape(B, 1, EMB),)


# speedup vs baseline: 1.1418x; 1.1418x over previous
import jax
import jax.numpy as jnp
from jax import lax
from jax.experimental import pallas as pl
from jax.experimental.pallas import tpu as pltpu
from jax.experimental.pallas import tpu_sc as plsc

VOCAB = 1000000
EMB = 32
B = 4096
L = 200

NC = 2
NS = 16
NW = NC * NS
BPW = B // NW
IPW = BPW * L
S0 = 96
S1 = L - S0
W = 128
PR = VOCAB // 4
CW = 512
NFULL = 999936 // CW
TAILBASE = NFULL * CW


def _repack_body(tt_hbm, tail_hbm, out_hbm, in_a, in_b, out_a, out_b,
                 sem_ia, sem_ib, sem_oa, sem_ob):
    c = lax.axis_index("c")
    s = lax.axis_index("s")
    wid = s * NC + c

    lane = lax.iota(jnp.int32, 16)
    lane32 = lane * EMB

    n_my = jnp.where(wid < NFULL - (NFULL // NW) * NW, NFULL // NW + 1,
                     NFULL // NW)

    def cbase(k):
        return (wid + jnp.minimum(k, n_my - 1) * NW) * CW

    def fire_in(k, buf, sem):
        pltpu.async_copy(tt_hbm.at[:, pl.ds(cbase(k), CW)], buf, sem)

    def wait_in(buf, sem):
        pltpu.make_async_copy(tt_hbm.at[:, pl.ds(0, CW)], buf, sem).wait()

    def fire_out(k, buf, sem):
        pltpu.async_copy(buf, out_hbm.at[pl.ds(cbase(k) * EMB, CW * EMB)], sem)

    def wait_out(buf, sem):
        pltpu.make_async_copy(buf, out_hbm.at[pl.ds(0, CW * EMB)], sem).wait()

    def shuffle(src, dst):
        def g_loop(g, _):
            dslice = dst.at[pl.ds(g * 512, 512)]
            v0 = g * 16
            for e in range(EMB):
                plsc.store_scatter(dslice, [lane32 + e], src[e, pl.ds(v0, 16)])
            return 0

        lax.fori_loop(0, CW // 16, g_loop, 0)

    fire_in(0, in_a, sem_ia)
    fire_in(1, in_b, sem_ib)

    def step(i, _):
        k = 2 * i
        wait_in(in_a, sem_ia)

        @pl.when(i > 0)
        def _():
            wait_out(out_a, sem_oa)

        shuffle(in_a, out_a)
        fire_out(k, out_a, sem_oa)
        fire_in(k + 2, in_a, sem_ia)

        wait_in(in_b, sem_ib)

        @pl.when(i > 0)
        def _():
            wait_out(out_b, sem_ob)

        shuffle(in_b, out_b)
        fire_out(k + 1, out_b, sem_ob)
        fire_in(k + 3, in_b, sem_ib)
        return 0

    lax.fori_loop(0, (n_my + 1) // 2, step, 0)

    wait_in(in_a, sem_ia)
    wait_in(in_b, sem_ib)
    wait_out(out_a, sem_oa)
    wait_out(out_b, sem_ob)

    @pl.when(wid == 17)
    def _():
        pltpu.sync_copy(tail_hbm, out_hbm.at[pl.ds(TAILBASE * EMB, 64 * EMB)])


def _gather_body(ids_hbm, tbl_hbm, out_hbm, idx_v, buf_a, buf_b, out_v, sem_a, sem_b):
    c = lax.axis_index("c")
    s = lax.axis_index("s")
    wid = s * NC + c

    pltpu.sync_copy(ids_hbm.at[pl.ds(wid * IPW, IPW)], idx_v)

    def fire(r, buf, sem):
        pltpu.async_copy(tbl_hbm.at[idx_v.at[pl.ds(L * r, S0)]],
                         buf.at[pl.ds(0, S0)], sem)
        pltpu.async_copy(tbl_hbm.at[idx_v.at[pl.ds(L * r + S0, S1)]],
                         buf.at[pl.ds(S0, S1)], sem)

    def drain(buf, sem):
        pltpu.make_async_copy(tbl_hbm.at[pl.ds(0, L)], buf, sem).wait()

    def accum(buf, r):
        zero = jnp.zeros((16,), jnp.float32)

        def body(j, carry):
            a0, a1 = carry
            a0 = a0 + buf[j, pl.ds(0, 16)]
            a1 = a1 + buf[j, pl.ds(16, 16)]
            return a0, a1

        a0, a1 = lax.fori_loop(0, L, body, (zero, zero), unroll=8)
        out_v[pl.ds(EMB * r, 16)] = a0
        out_v[pl.ds(EMB * r + 16, 16)] = a1

    fire(0, buf_a, sem_a)
    fire(1, buf_b, sem_b)

    def step(i, _):
        g = 2 * i
        drain(buf_a, sem_a)
        accum(buf_a, g)
        fire(jnp.minimum(g + 2, BPW - 1), buf_a, sem_a)
        drain(buf_b, sem_b)
        accum(buf_b, g + 1)
        fire(jnp.minimum(g + 3, BPW - 1), buf_b, sem_b)
        return 0

    lax.fori_loop(0, BPW // 2, step, 0)
    drain(buf_a, sem_a)
    drain(buf_b, sem_b)

    pltpu.sync_copy(out_v, out_hbm.at[pl.ds(wid * BPW * EMB, BPW * EMB)])


@jax.jit
def _encode(ids1, table_t, tail_packed):
    mesh = plsc.VectorSubcoreMesh(core_axis_name="c", subcore_axis_name="s")
    repack = pl.kernel(
        _repack_body,
        out_type=jax.ShapeDtypeStruct((PR * W,), jnp.float32),
        mesh=mesh,
        scratch_types=[
            pltpu.VMEM((EMB, CW), jnp.float32),
            pltpu.VMEM((EMB, CW), jnp.float32),
            pltpu.VMEM((CW * EMB,), jnp.float32),
            pltpu.VMEM((CW * EMB,), jnp.float32),
            pltpu.SemaphoreType.DMA,
            pltpu.SemaphoreType.DMA,
            pltpu.SemaphoreType.DMA,
            pltpu.SemaphoreType.DMA,
        ],
        name="repack",
        compiler_params=pltpu.CompilerParams(use_tc_tiling_on_sc=True,
                                             disable_bounds_checks=True,
                                             needs_layout_passes=False),
    )
    packed = repack(table_t, tail_packed)

    gather = pl.kernel(
        _gather_body,
        out_type=jax.ShapeDtypeStruct((B * EMB,), jnp.float32),
        mesh=mesh,
        scratch_types=[
            pltpu.VMEM((IPW,), jnp.int32),
            pltpu.VMEM((L, EMB), jnp.float32),
            pltpu.VMEM((L, EMB), jnp.float32),
            pltpu.VMEM((BPW * EMB,), jnp.float32),
            pltpu.SemaphoreType.DMA,
            pltpu.SemaphoreType.DMA,
        ],
        compiler_params=pltpu.CompilerParams(use_tc_tiling_on_sc=False),
    )
    return gather(ids1, packed.reshape(VOCAB, EMB))


def kernel(input_ids, embedding_table):
    ids1 = input_ids.astype(jnp.int32).reshape(B * L)
    tail_packed = embedding_table[VOCAB - 64:, :].reshape(64 * EMB)
    out = _encode(ids1, embedding_table.T, tail_packed)
    return (out.reshape(B, 1, EMB),)


# speedup vs baseline: 1.4292x; 1.2517x over previous
import jax
import jax.numpy as jnp
from jax import lax
from jax.experimental import pallas as pl
from jax.experimental.pallas import tpu as pltpu
from jax.experimental.pallas import tpu_sc as plsc

VOCAB = 1000000
EMB = 32
B = 4096
L = 200

NC = 2
NS = 16
NW = NC * NS
BPW = B // NW
IPW = BPW * L
S0 = 96
S1 = L - S0
W = 128
PR = VOCAB // 4
CW = 512
NFULL = 999936 // CW
TAILBASE = NFULL * CW


def _repack_body(tt_hbm, tail_hbm, out_hbm, in_a, in_b, out_a, out_b,
                 sem_ia, sem_ib, sem_oa, sem_ob):
    c = lax.axis_index("c")
    s = lax.axis_index("s")
    wid = s * NC + c

    lane = lax.iota(jnp.int32, 16)
    lane32 = lane * EMB

    n_my = jnp.where(wid < NFULL - (NFULL // NW) * NW, NFULL // NW + 1,
                     NFULL // NW)

    def cbase(k):
        return (wid + jnp.minimum(k, n_my - 1) * NW) * CW

    def fire_in(k, buf, sem):
        pltpu.async_copy(tt_hbm.at[:, pl.ds(cbase(k), CW)], buf, sem)

    def wait_in(buf, sem):
        pltpu.make_async_copy(tt_hbm.at[:, pl.ds(0, CW)], buf, sem).wait()

    def fire_out(k, buf, sem):
        pltpu.async_copy(buf, out_hbm.at[pl.ds(cbase(k) * EMB, CW * EMB)], sem)

    def wait_out(buf, sem):
        pltpu.make_async_copy(buf, out_hbm.at[pl.ds(0, CW * EMB)], sem).wait()

    idx_vecs = [lane32 + e for e in range(EMB)]

    def shuffle(src, dst):
        def g_loop(g, _):
            dslice = dst.at[pl.ds(g * 512, 512)]
            v0 = g * 16
            for e0 in range(0, EMB, 8):
                vals = [src[e0 + t, pl.ds(v0, 16)] for t in range(8)]
                for t in range(8):
                    plsc.store_scatter(dslice, [idx_vecs[e0 + t]], vals[t])
            return 0

        lax.fori_loop(0, CW // 16, g_loop, 0)

    fire_in(0, in_a, sem_ia)
    fire_in(1, in_b, sem_ib)

    def step(i, _):
        k = 2 * i
        wait_in(in_a, sem_ia)

        @pl.when(i > 0)
        def _():
            wait_out(out_a, sem_oa)

        shuffle(in_a, out_a)
        fire_out(k, out_a, sem_oa)
        fire_in(k + 2, in_a, sem_ia)

        wait_in(in_b, sem_ib)

        @pl.when(i > 0)
        def _():
            wait_out(out_b, sem_ob)

        shuffle(in_b, out_b)
        fire_out(k + 1, out_b, sem_ob)
        fire_in(k + 3, in_b, sem_ib)
        return 0

    lax.fori_loop(0, (n_my + 1) // 2, step, 0)

    wait_in(in_a, sem_ia)
    wait_in(in_b, sem_ib)
    wait_out(out_a, sem_oa)
    wait_out(out_b, sem_ob)

    @pl.when(wid == 17)
    def _():
        pltpu.sync_copy(tail_hbm, out_hbm.at[pl.ds(TAILBASE * EMB, 64 * EMB)])


def _gather_body(ids_hbm, tbl_hbm, out_hbm, idx_v, buf_a, buf_b, out_v, sem_a, sem_b):
    c = lax.axis_index("c")
    s = lax.axis_index("s")
    wid = s * NC + c

    pltpu.sync_copy(ids_hbm.at[pl.ds(wid * IPW, IPW)], idx_v)

    def fire(r, buf, sem):
        pltpu.async_copy(tbl_hbm.at[idx_v.at[pl.ds(L * r, S0)]],
                         buf.at[pl.ds(0, S0)], sem)
        pltpu.async_copy(tbl_hbm.at[idx_v.at[pl.ds(L * r + S0, S1)]],
                         buf.at[pl.ds(S0, S1)], sem)

    def drain(buf, sem):
        pltpu.make_async_copy(tbl_hbm.at[pl.ds(0, L)], buf, sem).wait()

    def accum(buf, r):
        zero = jnp.zeros((16,), jnp.float32)

        def body(j, carry):
            a0, a1 = carry
            a0 = a0 + buf[j, pl.ds(0, 16)]
            a1 = a1 + buf[j, pl.ds(16, 16)]
            return a0, a1

        a0, a1 = lax.fori_loop(0, L, body, (zero, zero), unroll=8)
        out_v[pl.ds(EMB * r, 16)] = a0
        out_v[pl.ds(EMB * r + 16, 16)] = a1

    fire(0, buf_a, sem_a)
    fire(1, buf_b, sem_b)

    def step(i, _):
        g = 2 * i
        drain(buf_a, sem_a)
        accum(buf_a, g)
        fire(jnp.minimum(g + 2, BPW - 1), buf_a, sem_a)
        drain(buf_b, sem_b)
        accum(buf_b, g + 1)
        fire(jnp.minimum(g + 3, BPW - 1), buf_b, sem_b)
        return 0

    lax.fori_loop(0, BPW // 2, step, 0)
    drain(buf_a, sem_a)
    drain(buf_b, sem_b)

    pltpu.sync_copy(out_v, out_hbm.at[pl.ds(wid * BPW * EMB, BPW * EMB)])


@jax.jit
def _encode(ids1, table_t, tail_packed):
    mesh = plsc.VectorSubcoreMesh(core_axis_name="c", subcore_axis_name="s")
    repack = pl.kernel(
        _repack_body,
        out_type=jax.ShapeDtypeStruct((PR * W,), jnp.float32),
        mesh=mesh,
        scratch_types=[
            pltpu.VMEM((EMB, CW), jnp.float32),
            pltpu.VMEM((EMB, CW), jnp.float32),
            pltpu.VMEM((CW * EMB,), jnp.float32),
            pltpu.VMEM((CW * EMB,), jnp.float32),
            pltpu.SemaphoreType.DMA,
            pltpu.SemaphoreType.DMA,
            pltpu.SemaphoreType.DMA,
            pltpu.SemaphoreType.DMA,
        ],
        name="repack",
        compiler_params=pltpu.CompilerParams(use_tc_tiling_on_sc=True,
                                             disable_bounds_checks=True,
                                             needs_layout_passes=False),
    )
    packed = repack(table_t, tail_packed)

    gather = pl.kernel(
        _gather_body,
        out_type=jax.ShapeDtypeStruct((B * EMB,), jnp.float32),
        mesh=mesh,
        scratch_types=[
            pltpu.VMEM((IPW,), jnp.int32),
            pltpu.VMEM((L, EMB), jnp.float32),
            pltpu.VMEM((L, EMB), jnp.float32),
            pltpu.VMEM((BPW * EMB,), jnp.float32),
            pltpu.SemaphoreType.DMA,
            pltpu.SemaphoreType.DMA,
        ],
        compiler_params=pltpu.CompilerParams(use_tc_tiling_on_sc=False),
    )
    return gather(ids1, packed.reshape(VOCAB, EMB))


def kernel(input_ids, embedding_table):
    ids1 = input_ids.astype(jnp.int32).reshape(B * L)
    tail_packed = embedding_table[VOCAB - 64:, :].reshape(64 * EMB)
    out = _encode(ids1, embedding_table.T, tail_packed)
    return (out.reshape(B, 1, EMB),)


# speedup vs baseline: 4.0392x; 2.8263x over previous
import jax
import jax.numpy as jnp
from jax import lax
from jax.experimental import pallas as pl
from jax.experimental.pallas import tpu as pltpu
from jax.experimental.pallas import tpu_sc as plsc

VOCAB = 1000000
EMB = 32
B = 4096
L = 200

NC = 2
NS = 16
NW = NC * NS
BPW = B // NW
IPW = BPW * L
S0 = 96
S1 = L - S0
W = 128
PR = VOCAB // 4
CW = 512
NFULL = 999936 // CW
TAILBASE = NFULL * CW


def _repack_body(tt_hbm, tail_hbm, out_hbm, in_a, in_b, out_a, out_b,
                 sem_ia, sem_ib, sem_oa, sem_ob):
    c = lax.axis_index("c")
    s = lax.axis_index("s")
    wid = s * NC + c

    lane = lax.iota(jnp.int32, 16)
    lane32 = lane * EMB

    n_my = jnp.where(wid < NFULL - (NFULL // NW) * NW, NFULL // NW + 1,
                     NFULL // NW)

    def cbase(k):
        return (wid + jnp.minimum(k, n_my - 1) * NW) * CW

    def fire_in(k, buf, sem):
        pltpu.async_copy(tt_hbm.at[:, pl.ds(cbase(k), CW)], buf, sem)

    def wait_in(buf, sem):
        pltpu.make_async_copy(tt_hbm.at[:, pl.ds(0, CW)], buf, sem).wait()

    def fire_out(k, buf, sem):
        pltpu.async_copy(buf, out_hbm.at[pl.ds(cbase(k) * EMB, CW * EMB)], sem)

    def wait_out(buf, sem):
        pltpu.make_async_copy(buf, out_hbm.at[pl.ds(0, CW * EMB)], sem).wait()

    cs = [(lane + k) & 15 for k in range(16)]
    out_idx = [c * EMB + lane for c in cs]

    def shuffle(src, dst):
        def g_loop(g, _):
            vl0 = g * 16
            csg = [vl0 + c for c in cs]
            for e0 in (0, 16):
                sslice = src.at[pl.ds(e0, 16)]
                dslice = dst.at[pl.ds(g * 512 + e0, 496)]
                for k0 in range(0, 16, 8):
                    vals = [plsc.load_gather(sslice, [lane, csg[k0 + t]])
                            for t in range(8)]
                    for t in range(8):
                        plsc.store_scatter(dslice, [out_idx[k0 + t]], vals[t])
            return 0

        lax.fori_loop(0, CW // 16, g_loop, 0)

    fire_in(0, in_a, sem_ia)
    fire_in(1, in_b, sem_ib)

    def step(i, _):
        k = 2 * i
        wait_in(in_a, sem_ia)

        @pl.when(i > 0)
        def _():
            wait_out(out_a, sem_oa)

        shuffle(in_a, out_a)
        fire_out(k, out_a, sem_oa)
        fire_in(k + 2, in_a, sem_ia)

        wait_in(in_b, sem_ib)

        @pl.when(i > 0)
        def _():
            wait_out(out_b, sem_ob)

        shuffle(in_b, out_b)
        fire_out(k + 1, out_b, sem_ob)
        fire_in(k + 3, in_b, sem_ib)
        return 0

    lax.fori_loop(0, (n_my + 1) // 2, step, 0)

    wait_in(in_a, sem_ia)
    wait_in(in_b, sem_ib)
    wait_out(out_a, sem_oa)
    wait_out(out_b, sem_ob)

    @pl.when(wid == 17)
    def _():
        pltpu.sync_copy(tail_hbm, out_hbm.at[pl.ds(TAILBASE * EMB, 64 * EMB)])


def _gather_body(ids_hbm, tbl_hbm, out_hbm, idx_v, buf_a, buf_b, out_v, sem_a, sem_b):
    c = lax.axis_index("c")
    s = lax.axis_index("s")
    wid = s * NC + c

    pltpu.sync_copy(ids_hbm.at[pl.ds(wid * IPW, IPW)], idx_v)

    def fire(r, buf, sem):
        pltpu.async_copy(tbl_hbm.at[idx_v.at[pl.ds(L * r, S0)]],
                         buf.at[pl.ds(0, S0)], sem)
        pltpu.async_copy(tbl_hbm.at[idx_v.at[pl.ds(L * r + S0, S1)]],
                         buf.at[pl.ds(S0, S1)], sem)

    def drain(buf, sem):
        pltpu.make_async_copy(tbl_hbm.at[pl.ds(0, L)], buf, sem).wait()

    def accum(buf, r):
        zero = jnp.zeros((16,), jnp.float32)

        def body(j, carry):
            a0, a1 = carry
            a0 = a0 + buf[j, pl.ds(0, 16)]
            a1 = a1 + buf[j, pl.ds(16, 16)]
            return a0, a1

        a0, a1 = lax.fori_loop(0, L, body, (zero, zero), unroll=8)
        out_v[pl.ds(EMB * r, 16)] = a0
        out_v[pl.ds(EMB * r + 16, 16)] = a1

    fire(0, buf_a, sem_a)
    fire(1, buf_b, sem_b)

    def step(i, _):
        g = 2 * i
        drain(buf_a, sem_a)
        accum(buf_a, g)
        fire(jnp.minimum(g + 2, BPW - 1), buf_a, sem_a)
        drain(buf_b, sem_b)
        accum(buf_b, g + 1)
        fire(jnp.minimum(g + 3, BPW - 1), buf_b, sem_b)
        return 0

    lax.fori_loop(0, BPW // 2, step, 0)
    drain(buf_a, sem_a)
    drain(buf_b, sem_b)

    pltpu.sync_copy(out_v, out_hbm.at[pl.ds(wid * BPW * EMB, BPW * EMB)])


@jax.jit
def _encode(ids1, table_t, tail_packed):
    mesh = plsc.VectorSubcoreMesh(core_axis_name="c", subcore_axis_name="s")
    repack = pl.kernel(
        _repack_body,
        out_type=jax.ShapeDtypeStruct((PR * W,), jnp.float32),
        mesh=mesh,
        scratch_types=[
            pltpu.VMEM((EMB, CW), jnp.float32),
            pltpu.VMEM((EMB, CW), jnp.float32),
            pltpu.VMEM((CW * EMB,), jnp.float32),
            pltpu.VMEM((CW * EMB,), jnp.float32),
            pltpu.SemaphoreType.DMA,
            pltpu.SemaphoreType.DMA,
            pltpu.SemaphoreType.DMA,
            pltpu.SemaphoreType.DMA,
        ],
        name="repack",
        compiler_params=pltpu.CompilerParams(use_tc_tiling_on_sc=True,
                                             disable_bounds_checks=True,
                                             needs_layout_passes=False),
    )
    packed = repack(table_t, tail_packed)

    gather = pl.kernel(
        _gather_body,
        out_type=jax.ShapeDtypeStruct((B * EMB,), jnp.float32),
        mesh=mesh,
        scratch_types=[
            pltpu.VMEM((IPW,), jnp.int32),
            pltpu.VMEM((L, EMB), jnp.float32),
            pltpu.VMEM((L, EMB), jnp.float32),
            pltpu.VMEM((BPW * EMB,), jnp.float32),
            pltpu.SemaphoreType.DMA,
            pltpu.SemaphoreType.DMA,
        ],
        compiler_params=pltpu.CompilerParams(use_tc_tiling_on_sc=False),
    )
    return gather(ids1, packed.reshape(VOCAB, EMB))


def kernel(input_ids, embedding_table):
    ids1 = input_ids.astype(jnp.int32).reshape(B * L)
    tail_packed = embedding_table[VOCAB - 64:, :].reshape(64 * EMB)
    out = _encode(ids1, embedding_table.T, tail_packed)
    return (out.reshape(B, 1, EMB),)


# speedup vs baseline: 4.0462x; 1.0017x over previous
import jax
import jax.numpy as jnp
from jax import lax
from jax.experimental import pallas as pl
from jax.experimental.pallas import tpu as pltpu
from jax.experimental.pallas import tpu_sc as plsc

VOCAB = 1000000
EMB = 32
B = 4096
L = 200

NC = 2
NS = 16
NW = NC * NS
BPW = B // NW
IPW = BPW * L
S0 = 96
S1 = L - S0
W = 128
PR = VOCAB // 4
CW = 512
NFULL = 999936 // CW
TAILBASE = NFULL * CW


def _repack_body(tt_hbm, tail_hbm, out_hbm, in_a, in_b, out_a, out_b,
                 sem_ia, sem_ib, sem_oa, sem_ob):
    c = lax.axis_index("c")
    s = lax.axis_index("s")
    wid = s * NC + c

    lane = lax.iota(jnp.int32, 16)

    n_my = jnp.where(wid < NFULL - (NFULL // NW) * NW, NFULL // NW + 1,
                     NFULL // NW)

    def cbase(k):
        return (wid + jnp.minimum(k, n_my - 1) * NW) * CW

    def fire_in(k, buf, sem):
        pltpu.async_copy(tt_hbm.at[:, pl.ds(cbase(k), CW)], buf, sem)

    def wait_in(buf, sem):
        pltpu.make_async_copy(tt_hbm.at[:, pl.ds(0, CW)], buf, sem).wait()

    def fire_out(k, buf, sem):
        pltpu.async_copy(buf, out_hbm.at[pl.ds(cbase(k) * EMB, CW * EMB)], sem)

    def wait_out(buf, sem):
        pltpu.make_async_copy(buf, out_hbm.at[pl.ds(0, CW * EMB)], sem).wait()

    cs = [(lane + k) & 15 for k in range(16)]
    out_idx = [c * EMB + lane for c in cs]

    def shuffle(src, dst):
        def g_loop(g, _):
            vl0 = g * 16
            csg = [vl0 + c for c in cs]
            for e0 in (0, 16):
                sslice = src.at[pl.ds(e0, 16)]
                dslice = dst.at[pl.ds(g * 512 + e0, 496)]
                for k0 in range(0, 16, 8):
                    vals = [plsc.load_gather(sslice, [lane, csg[k0 + t]])
                            for t in range(8)]
                    for t in range(8):
                        plsc.store_scatter(dslice, [out_idx[k0 + t]], vals[t])
            return 0

        lax.fori_loop(0, CW // 16, g_loop, 0)

    fire_in(0, in_a, sem_ia)
    fire_in(1, in_b, sem_ib)

    def step(i, _):
        k = 2 * i
        wait_in(in_a, sem_ia)

        @pl.when(i > 0)
        def _():
            wait_out(out_a, sem_oa)

        shuffle(in_a, out_a)
        fire_out(k, out_a, sem_oa)
        fire_in(k + 2, in_a, sem_ia)

        wait_in(in_b, sem_ib)

        @pl.when(i > 0)
        def _():
            wait_out(out_b, sem_ob)

        shuffle(in_b, out_b)
        fire_out(k + 1, out_b, sem_ob)
        fire_in(k + 3, in_b, sem_ib)
        return 0

    lax.fori_loop(0, (n_my + 1) // 2, step, 0)

    wait_in(in_a, sem_ia)
    wait_in(in_b, sem_ib)
    wait_out(out_a, sem_oa)
    wait_out(out_b, sem_ob)

    @pl.when(wid == 17)
    def _():
        pltpu.sync_copy(tail_hbm, out_hbm.at[pl.ds(TAILBASE * EMB, 64 * EMB)])


def _gather_body(ids_hbm, tbl_hbm, out_hbm, idx_v, buf_a, buf_b, out_v, sem_a, sem_b):
    c = lax.axis_index("c")
    s = lax.axis_index("s")
    wid = s * NC + c

    pltpu.sync_copy(ids_hbm.at[pl.ds(wid * IPW, IPW)], idx_v)

    def fire(r, buf, sem):
        pltpu.async_copy(tbl_hbm.at[idx_v.at[pl.ds(L * r, S0)]],
                         buf.at[pl.ds(0, S0)], sem)
        pltpu.async_copy(tbl_hbm.at[idx_v.at[pl.ds(L * r + S0, S1)]],
                         buf.at[pl.ds(S0, S1)], sem)

    def drain(buf, sem):
        pltpu.make_async_copy(tbl_hbm.at[pl.ds(0, L)], buf, sem).wait()

    def accum(buf, r):
        zero = jnp.zeros((16,), jnp.float32)

        def body(j, carry):
            a0, a1 = carry
            a0 = a0 + buf[j, pl.ds(0, 16)]
            a1 = a1 + buf[j, pl.ds(16, 16)]
            return a0, a1

        a0, a1 = lax.fori_loop(0, L, body, (zero, zero), unroll=8)
        out_v[pl.ds(EMB * r, 16)] = a0
        out_v[pl.ds(EMB * r + 16, 16)] = a1

    fire(0, buf_a, sem_a)
    fire(1, buf_b, sem_b)

    def step(i, _):
        g = 2 * i
        drain(buf_a, sem_a)
        accum(buf_a, g)
        fire(jnp.minimum(g + 2, BPW - 1), buf_a, sem_a)
        drain(buf_b, sem_b)
        accum(buf_b, g + 1)
        fire(jnp.minimum(g + 3, BPW - 1), buf_b, sem_b)
        return 0

    lax.fori_loop(0, BPW // 2, step, 0)
    drain(buf_a, sem_a)
    drain(buf_b, sem_b)

    pltpu.sync_copy(out_v, out_hbm.at[pl.ds(wid * BPW * EMB, BPW * EMB)])


@jax.jit
def _encode(ids1, table_t, tail_packed):
    mesh = plsc.VectorSubcoreMesh(core_axis_name="c", subcore_axis_name="s")
    repack = pl.kernel(
        _repack_body,
        out_type=jax.ShapeDtypeStruct((PR * W,), jnp.float32),
        mesh=mesh,
        scratch_types=[
            pltpu.VMEM((EMB, CW), jnp.float32),
            pltpu.VMEM((EMB, CW), jnp.float32),
            pltpu.VMEM((CW * EMB,), jnp.float32),
            pltpu.VMEM((CW * EMB,), jnp.float32),
            pltpu.SemaphoreType.DMA,
            pltpu.SemaphoreType.DMA,
            pltpu.SemaphoreType.DMA,
            pltpu.SemaphoreType.DMA,
        ],
        name="repack",
        compiler_params=pltpu.CompilerParams(use_tc_tiling_on_sc=True,
                                             disable_bounds_checks=True,
                                             needs_layout_passes=False),
    )
    packed = repack(table_t, tail_packed)

    gather = pl.kernel(
        _gather_body,
        out_type=jax.ShapeDtypeStruct((B * EMB,), jnp.float32),
        mesh=mesh,
        scratch_types=[
            pltpu.VMEM((IPW,), jnp.int32),
            pltpu.VMEM((L, EMB), jnp.float32),
            pltpu.VMEM((L, EMB), jnp.float32),
            pltpu.VMEM((BPW * EMB,), jnp.float32),
            pltpu.SemaphoreType.DMA,
            pltpu.SemaphoreType.DMA,
        ],
        compiler_params=pltpu.CompilerParams(use_tc_tiling_on_sc=False),
    )
    return gather(ids1, packed.reshape(VOCAB, EMB))


def kernel(input_ids, embedding_table):
    ids1 = input_ids.astype(jnp.int32).reshape(B * L)
    tail_packed = embedding_table[VOCAB - 64:, :].reshape(64 * EMB)
    out = _encode(ids1, embedding_table.T, tail_packed)
    return (out.reshape(B, 1, EMB),)
